# Initial kernel scaffold; baseline (speedup 1.0000x reference)
import jax
import jax.numpy as jnp
from jax.experimental import pallas as pl


def kernel(id, center_embed):
    raise NotImplementedError("write your pallas kernel here")


# speedup vs baseline: 1.0333x; 1.0333x over previous
import functools

import jax
import jax.numpy as jnp
from jax import lax
from jax.experimental import pallas as pl
from jax.experimental.pallas import tpu as pltpu
from jax.experimental.pallas import tpu_sc as plsc

_VOCAB = 1000000
_EMBED_DIM = 64
_BATCH = 16384
_NBUF = 4
_LANES = 16


def _gather_t(idx, tab_t):
    info = plsc.get_sparse_core_info()
    num_workers = info.num_cores * info.num_subcores
    b_per_w = _BATCH // num_workers

    mesh = plsc.VectorSubcoreMesh(core_axis_name="c", subcore_axis_name="s")

    @functools.partial(
        pl.kernel,
        mesh=mesh,
        out_type=jax.ShapeDtypeStruct((_BATCH, _EMBED_DIM), jnp.float32),
        scratch_types=[
            pltpu.SMEM((b_per_w,), jnp.int32),
            pltpu.VMEM((b_per_w,), jnp.int32),
            pltpu.VMEM((b_per_w, _EMBED_DIM), jnp.float32),
        ]
        + [pltpu.VMEM((_EMBED_DIM, 128), jnp.float32) for _ in range(_NBUF)]
        + [pltpu.SemaphoreType.DMA for _ in range(_NBUF)]
        + [pltpu.SemaphoreType.DMA],
        compiler_params=pltpu.CompilerParams(needs_layout_passes=False),
    )
    def gather_kernel(tab_hbm, idx_hbm, out_hbm, idx_s, idx_v, rows_v,
                      blk0, blk1, blk2, blk3, sem0, sem1, sem2, sem3, sem_i):
        blks = (blk0, blk1, blk2, blk3)
        sems = (sem0, sem1, sem2, sem3)
        wid = lax.axis_index("s") * info.num_cores + lax.axis_index("c")
        base = wid * b_per_w
        pltpu.async_copy(idx_hbm.at[pl.ds(base, b_per_w)], idx_v, sem_i).wait()

        def spill_chunk(g, carry):
            chunk = idx_v[pl.ds(g * _LANES, _LANES)]
            for k in range(_LANES):
                idx_s[g * _LANES + k] = chunk[k]
            return carry

        lax.fori_loop(0, b_per_w // _LANES, spill_chunk, 0)

        def fire(i, slot):
            v = idx_s[jnp.minimum(i, b_per_w - 1)]
            bc = pl.multiple_of((v >> 7) * 128, 128)
            pltpu.make_async_copy(
                tab_hbm.at[:, pl.ds(bc, 128)], blks[slot], sems[slot]
            ).start()

        for r in range(_NBUF):
            fire(r, r)

        lane = lax.iota(jnp.int32, _LANES)

        def body(g, carry):
            for r in range(_NBUF):
                i = g * _NBUF + r
                pltpu.make_async_copy(
                    tab_hbm.at[:, pl.ds(0, 128)], blks[r], sems[r]
                ).wait()
                v = idx_s[i]
                lidx = jnp.full((_LANES,), v & 127, jnp.int32)
                for k in range(_EMBED_DIM // _LANES):
                    xs = plsc.load_gather(blks[r], [lane + (k * _LANES), lidx])
                    rows_v[i, pl.ds(k * _LANES, _LANES)] = xs
                fire(i + _NBUF, r)
            return carry

        lax.fori_loop(0, b_per_w // _NBUF, body, 0)
        for r in range(_NBUF):
            pltpu.make_async_copy(
                tab_hbm.at[:, pl.ds(0, 128)], blks[r], sems[r]
            ).wait()
        pltpu.sync_copy(rows_v, out_hbm.at[pl.ds(base, b_per_w)])

    return gather_kernel(tab_t, idx)


def kernel(id, center_embed):
    idx = id.astype(jnp.int32)
    return _gather_t(idx, center_embed.T)
